# baseline (speedup 1.0000x reference)
import functools

import jax
import jax.numpy as jnp
from jax import lax
from jax.experimental import pallas as pl
from jax.experimental.pallas import tpu as pltpu
from jax.experimental.pallas import tpu_sc as plsc

N_LOCS = 640000
N_CLUSTERS = 10000
N_FOV = 16
HID = 64
N_CLASSES = 4

R1 = 5120
CHUNK = 512
NW = 32
CPT = 320
NC_PAD = NW * CPT
TROWS = CPT + 16


def _mlp_body(xt_ref, w1_ref, b1_ref, w2_ref, b2_ref, o_ref):
    h = lax.dot_general(xt_ref[...], w1_ref[...],
                        (((0,), (0,)), ((), ())),
                        preferred_element_type=jnp.float32)
    h = jnp.maximum(h + b1_ref[0:1, :], 0.0).astype(jnp.bfloat16)
    h = jnp.dot(h, w2_ref[...], preferred_element_type=jnp.float32)
    o_ref[...] = jnp.maximum(h + b2_ref[0:1, :], 0.0)


def _run_mlp(xt, w1b, b1r, w2b, b2r):
    grid = (N_LOCS // R1,)
    return pl.pallas_call(
        _mlp_body,
        grid=grid,
        in_specs=[
            pl.BlockSpec((7, R1), lambda i: (0, i)),
            pl.BlockSpec((7, HID), lambda i: (0, 0)),
            pl.BlockSpec((8, HID), lambda i: (0, 0)),
            pl.BlockSpec((HID, HID), lambda i: (0, 0)),
            pl.BlockSpec((8, HID), lambda i: (0, 0)),
        ],
        out_specs=pl.BlockSpec((R1, HID), lambda i: (i, 0)),
        out_shape=jax.ShapeDtypeStruct((N_LOCS, HID), jnp.float32),
    )(xt, w1b, b1r, w2b, b2r)


def _segmax_body(h2_hbm, ids_hbm, bounds_hbm, out_hbm, bounds_v, ids_v, rows_v, table_v, acc_v):
    w = lax.axis_index("s") * 2 + lax.axis_index("c")
    pltpu.sync_copy(bounds_hbm, bounds_v)
    bv = bounds_v[pl.ds(w, 16)]
    row_lo = bv[0]
    row_hi = bv[1]
    base = lax.bitwise_and(row_lo, jnp.int32(-16))
    c_lo = w * CPT

    def _zero(i, carry):
        for f in range(HID // 16):
            table_v[i, pl.ds(16 * f, 16)] = jnp.zeros((16,), jnp.float32)
        return carry

    lax.fori_loop(0, TROWS, _zero, 0)

    nk = (row_hi - base + (CHUNK - 1)) // CHUNK
    zero16 = jnp.zeros((16,), jnp.float32)

    for f in range(HID // 16):
        acc_v[0, pl.ds(16 * f, 16)] = zero16

    def _flush(prev):
        for f in range(HID // 16):
            sl = pl.ds(16 * f, 16)
            table_v[prev, sl] = jnp.maximum(table_v[prev, sl], acc_v[0, sl])

    def _chunk(k, carry):
        s = pl.multiple_of(jnp.minimum(base + k * CHUNK, N_LOCS - CHUNK), 16)
        pltpu.sync_copy(ids_hbm.at[pl.ds(s, CHUNK)], ids_v)
        pltpu.sync_copy(h2_hbm.at[pl.ds(s, CHUNK), :], rows_v)

        def _grp(g, prev):
            r0 = g * 16
            idvec = ids_v[pl.ds(r0, 16)]
            rel_f = idvec[0] - c_lo
            rel1_f = jnp.where(
                jnp.logical_and(rel_f >= 0, rel_f < CPT), rel_f, CPT)
            uniform = jnp.logical_and(idvec[0] == idvec[15], rel1_f == prev)

            def _fast(_):
                for f in range(HID // 16):
                    sl = pl.ds(16 * f, 16)
                    vs = [rows_v[r0 + j, sl] for j in range(16)]
                    while len(vs) > 1:
                        nxt = [jnp.maximum(vs[i], vs[i + 1])
                               for i in range(0, len(vs) - 1, 2)]
                        if len(vs) % 2:
                            nxt.append(vs[-1])
                        vs = nxt
                    acc_v[0, sl] = jnp.maximum(acc_v[0, sl], vs[0])
                return prev

            def _slow(_):
                p = prev
                for j in range(16):
                    rel = idvec[j] - c_lo
                    rel1 = jnp.where(
                        jnp.logical_and(rel >= 0, rel < CPT), rel, CPT)
                    changed = rel1 != p

                    @pl.when(changed)
                    def _(p=p):
                        _flush(p)
                        for f in range(HID // 16):
                            acc_v[0, pl.ds(16 * f, 16)] = zero16

                    for f in range(HID // 16):
                        sl = pl.ds(16 * f, 16)
                        acc_v[0, sl] = jnp.maximum(
                            acc_v[0, sl], rows_v[r0 + j, sl])
                    p = rel1
                return p

            return lax.cond(uniform, _fast, _slow, None)

        return lax.fori_loop(0, CHUNK // 16, _grp, carry)

    prev = lax.fori_loop(0, nk, _chunk, jnp.int32(CPT))
    _flush(prev)
    pltpu.sync_copy(table_v.at[pl.ds(0, CPT), :], out_hbm.at[pl.ds(c_lo, CPT), :])


def _run_segmax(h2, ids, bounds):
    mesh = plsc.VectorSubcoreMesh(core_axis_name="c", subcore_axis_name="s")
    f = functools.partial(
        pl.kernel,
        mesh=mesh,
        out_type=jax.ShapeDtypeStruct((NC_PAD, HID), jnp.float32),
        scratch_types=[
            pltpu.VMEM((48,), jnp.int32),
            pltpu.VMEM((CHUNK,), jnp.int32),
            pltpu.VMEM((CHUNK, HID), jnp.float32),
            pltpu.VMEM((TROWS, HID), jnp.float32),
            pltpu.VMEM((8, HID), jnp.float32),
        ],
    )(_segmax_body)
    return f(h2, ids, bounds)


def _head_body(xc_ref, w3_ref, b3_ref, cb_ref, o_ref):
    xc3 = jnp.dot(xc_ref[...], w3_ref[...], preferred_element_type=jnp.float32)
    xc3 = xc3 + b3_ref[0:1, :]
    cb = cb_ref[0:1, :]
    iot = lax.broadcasted_iota(jnp.int32, (N_FOV, NC_PAD), 0)
    onehot = (iot == cb).astype(jnp.float32)
    sums = jnp.dot(onehot, xc3, preferred_element_type=jnp.float32)
    counts = jnp.sum(onehot, axis=1, keepdims=True)
    xfov = sums / jnp.maximum(counts, 1.0)
    logits = xfov[:, 0:N_CLASSES]
    m = jnp.max(logits, axis=1, keepdims=True)
    ls = (logits - m) - jnp.log(jnp.sum(jnp.exp(logits - m), axis=1, keepdims=True))
    o_ref[...] = ls


def _run_head(xc, w3p, b3r, cb2):
    return pl.pallas_call(
        _head_body,
        in_specs=[
            pl.BlockSpec((NC_PAD, HID), lambda: (0, 0)),
            pl.BlockSpec((HID, 8), lambda: (0, 0)),
            pl.BlockSpec((8, 8), lambda: (0, 0)),
            pl.BlockSpec((8, NC_PAD), lambda: (0, 0)),
        ],
        out_specs=pl.BlockSpec((N_FOV, N_CLASSES), lambda: (0, 0)),
        out_shape=jax.ShapeDtypeStruct((N_FOV, N_CLASSES), jnp.float32),
    )(xc, w3p, b3r, cb2)


def kernel(x_locs, pos_locs, cluster_id, cluster_batch, W1, b1, W2, b2, W3, b3):
    targets = jnp.minimum(jnp.arange(NW + 1, dtype=jnp.int32) * CPT, N_CLUSTERS)
    bounds = jnp.searchsorted(cluster_id, targets,
                              method="compare_all").astype(jnp.int32)
    bounds = jnp.pad(bounds, (0, 48 - (NW + 1)))

    xt = jnp.concatenate([x_locs.T, pos_locs.T], axis=0).astype(jnp.bfloat16)

    w1b = W1.astype(jnp.bfloat16)
    b1r = jnp.broadcast_to(b1[None, :], (8, HID))
    w2b = W2.astype(jnp.bfloat16)
    b2r = jnp.broadcast_to(b2[None, :], (8, HID))
    w3p = jnp.pad(W3, ((0, 0), (0, 8 - N_CLASSES)))
    b3r = jnp.broadcast_to(jnp.pad(b3, (0, 8 - N_CLASSES))[None, :], (8, 8))
    cb_pad = jnp.concatenate(
        [cluster_batch, jnp.full((NC_PAD - N_CLUSTERS,), -1, jnp.int32)])
    cb2 = jnp.broadcast_to(cb_pad[None, :], (8, NC_PAD))

    h2 = _run_mlp(xt, w1b, b1r, w2b, b2r)
    xc = _run_segmax(h2, cluster_id, bounds)
    return _run_head(xc, w3p, b3r, cb2)


# speedup vs baseline: 1.1238x; 1.1238x over previous
import functools

import jax
import jax.numpy as jnp
from jax import lax
from jax.experimental import pallas as pl
from jax.experimental.pallas import tpu as pltpu
from jax.experimental.pallas import tpu_sc as plsc

N_LOCS = 640000
N_CLUSTERS = 10000
N_FOV = 16
HID = 64
N_CLASSES = 4

R1 = 5120
CHUNK = 512
NW = 32
CPT = 320
NC_PAD = NW * CPT
TROWS = CPT + 16


def _mlp_body(xt_ref, w1_ref, b1_ref, w2_ref, b2_ref, o_ref):
    h = lax.dot_general(xt_ref[...], w1_ref[...],
                        (((0,), (0,)), ((), ())),
                        preferred_element_type=jnp.float32)
    h = jnp.maximum(h + b1_ref[0:1, :], 0.0).astype(jnp.bfloat16)
    h = jnp.dot(h, w2_ref[...], preferred_element_type=jnp.float32)
    o_ref[...] = jnp.maximum(h + b2_ref[0:1, :], 0.0)


def _run_mlp(xt, w1b, b1r, w2b, b2r):
    grid = (N_LOCS // R1,)
    return pl.pallas_call(
        _mlp_body,
        grid=grid,
        in_specs=[
            pl.BlockSpec((7, R1), lambda i: (0, i)),
            pl.BlockSpec((7, HID), lambda i: (0, 0)),
            pl.BlockSpec((8, HID), lambda i: (0, 0)),
            pl.BlockSpec((HID, HID), lambda i: (0, 0)),
            pl.BlockSpec((8, HID), lambda i: (0, 0)),
        ],
        out_specs=pl.BlockSpec((R1, HID), lambda i: (i, 0)),
        out_shape=jax.ShapeDtypeStruct((N_LOCS, HID), jnp.float32),
    )(xt, w1b, b1r, w2b, b2r)


def _segmax_body(h2_hbm, ids_hbm, bounds_hbm, out_hbm, bounds_v, ids_v, rows_v, table_v):
    w = lax.axis_index("s") * 2 + lax.axis_index("c")
    pltpu.sync_copy(bounds_hbm, bounds_v)
    bv = bounds_v[pl.ds(w, 16)]
    row_lo = bv[0]
    row_hi = bv[1]
    base = lax.bitwise_and(row_lo, jnp.int32(-16))
    c_lo = w * CPT

    def _zero(i, carry):
        for f in range(HID // 16):
            table_v[i, pl.ds(16 * f, 16)] = jnp.zeros((16,), jnp.float32)
        return carry

    lax.fori_loop(0, TROWS, _zero, 0)

    nk = (row_hi - base + (CHUNK - 1)) // CHUNK
    zero16 = jnp.zeros((16,), jnp.float32)

    def _flush(prev, accs):
        for f in range(HID // 16):
            sl = pl.ds(16 * f, 16)
            table_v[prev, sl] = jnp.maximum(table_v[prev, sl], accs[f])

    def _chunk(k, carry):
        s = pl.multiple_of(jnp.minimum(base + k * CHUNK, N_LOCS - CHUNK), 16)
        pltpu.sync_copy(ids_hbm.at[pl.ds(s, CHUNK)], ids_v)
        pltpu.sync_copy(h2_hbm.at[pl.ds(s, CHUNK), :], rows_v)

        def _grp(g, gcarry):
            prev, accs = gcarry
            idvec = ids_v[pl.ds(g * 16, 16)]
            for j in range(16):
                rel = idvec[j] - c_lo
                rel1 = jnp.where(
                    jnp.logical_and(rel >= 0, rel < CPT), rel, CPT)
                changed = rel1 != prev

                @pl.when(changed)
                def _(prev=prev, accs=accs):
                    _flush(prev, accs)

                r = g * 16 + j
                accs = tuple(
                    jnp.maximum(jnp.where(changed, zero16, accs[f]),
                                rows_v[r, pl.ds(16 * f, 16)])
                    for f in range(HID // 16))
                prev = rel1
            return prev, accs

        return lax.fori_loop(0, CHUNK // 16, _grp, carry)

    prev, accs = lax.fori_loop(0, nk, _chunk,
                               (jnp.int32(CPT), (zero16,) * (HID // 16)))
    _flush(prev, accs)
    pltpu.sync_copy(table_v.at[pl.ds(0, CPT), :], out_hbm.at[pl.ds(c_lo, CPT), :])


def _run_segmax(h2, ids, bounds):
    mesh = plsc.VectorSubcoreMesh(core_axis_name="c", subcore_axis_name="s")
    f = functools.partial(
        pl.kernel,
        mesh=mesh,
        out_type=jax.ShapeDtypeStruct((NC_PAD, HID), jnp.float32),
        scratch_types=[
            pltpu.VMEM((48,), jnp.int32),
            pltpu.VMEM((CHUNK,), jnp.int32),
            pltpu.VMEM((CHUNK, HID), jnp.float32),
            pltpu.VMEM((TROWS, HID), jnp.float32),
        ],
    )(_segmax_body)
    return f(h2, ids, bounds)


def _head_body(xc_ref, w3_ref, b3_ref, cb_ref, o_ref):
    xc3 = jnp.dot(xc_ref[...], w3_ref[...], preferred_element_type=jnp.float32)
    xc3 = xc3 + b3_ref[0:1, :]
    cb = cb_ref[0:1, :]
    iot = lax.broadcasted_iota(jnp.int32, (N_FOV, NC_PAD), 0)
    onehot = (iot == cb).astype(jnp.float32)
    sums = jnp.dot(onehot, xc3, preferred_element_type=jnp.float32)
    counts = jnp.sum(onehot, axis=1, keepdims=True)
    xfov = sums / jnp.maximum(counts, 1.0)
    logits = xfov[:, 0:N_CLASSES]
    m = jnp.max(logits, axis=1, keepdims=True)
    ls = (logits - m) - jnp.log(jnp.sum(jnp.exp(logits - m), axis=1, keepdims=True))
    o_ref[...] = ls


def _run_head(xc, w3p, b3r, cb2):
    return pl.pallas_call(
        _head_body,
        in_specs=[
            pl.BlockSpec((NC_PAD, HID), lambda: (0, 0)),
            pl.BlockSpec((HID, 8), lambda: (0, 0)),
            pl.BlockSpec((8, 8), lambda: (0, 0)),
            pl.BlockSpec((8, NC_PAD), lambda: (0, 0)),
        ],
        out_specs=pl.BlockSpec((N_FOV, N_CLASSES), lambda: (0, 0)),
        out_shape=jax.ShapeDtypeStruct((N_FOV, N_CLASSES), jnp.float32),
    )(xc, w3p, b3r, cb2)


def kernel(x_locs, pos_locs, cluster_id, cluster_batch, W1, b1, W2, b2, W3, b3):
    targets = jnp.minimum(jnp.arange(NW + 1, dtype=jnp.int32) * CPT, N_CLUSTERS)
    bounds = jnp.searchsorted(cluster_id, targets,
                              method="compare_all").astype(jnp.int32)
    bounds = jnp.pad(bounds, (0, 48 - (NW + 1)))

    xt = jnp.concatenate([x_locs.T, pos_locs.T], axis=0).astype(jnp.bfloat16)

    w1b = W1.astype(jnp.bfloat16)
    b1r = jnp.broadcast_to(b1[None, :], (8, HID))
    w2b = W2.astype(jnp.bfloat16)
    b2r = jnp.broadcast_to(b2[None, :], (8, HID))
    w3p = jnp.pad(W3, ((0, 0), (0, 8 - N_CLASSES)))
    b3r = jnp.broadcast_to(jnp.pad(b3, (0, 8 - N_CLASSES))[None, :], (8, 8))
    cb_pad = jnp.concatenate(
        [cluster_batch, jnp.full((NC_PAD - N_CLUSTERS,), -1, jnp.int32)])
    cb2 = jnp.broadcast_to(cb_pad[None, :], (8, NC_PAD))

    h2 = _run_mlp(xt, w1b, b1r, w2b, b2r)
    xc = _run_segmax(h2, cluster_id, bounds)
    return _run_head(xc, w3p, b3r, cb2)
